# Initial kernel scaffold; baseline (speedup 1.0000x reference)
import jax
import jax.numpy as jnp
from jax.experimental import pallas as pl


def kernel(x, weight):
    raise NotImplementedError("write your pallas kernel here")


# speedup vs baseline: 1.6849x; 1.6849x over previous
import functools

import jax
import jax.numpy as jnp
from jax import lax
from jax.experimental import pallas as pl
from jax.experimental.pallas import tpu as pltpu
from jax.experimental.pallas import tpu_sc as plsc

_D = 64
_B = 16384 * 50
_NC = 2
_NS = 16
_NW = _NC * _NS
_BPW = _B // _NW
_CH = 128
_NCH = _BPW // _CH

_mesh = plsc.VectorSubcoreMesh(core_axis_name="c", subcore_axis_name="s")


@functools.partial(
    pl.kernel,
    mesh=_mesh,
    out_type=jax.ShapeDtypeStruct((_B, _D), jnp.float32),
    scratch_types=[
        pltpu.VMEM((_BPW,), jnp.int32),
        pltpu.VMEM((_CH, _D), jnp.float32),
        pltpu.SemaphoreType.DMA,
    ],
    compiler_params=pltpu.CompilerParams(use_tc_tiling_on_sc=False),
)
def _emb_lookup(w_hbm, x_hbm, out_hbm, idx_v, rows, sem):
    wid = lax.axis_index("s") * _NC + lax.axis_index("c")
    base = wid * _BPW
    pltpu.sync_copy(x_hbm.at[pl.ds(base, _BPW)], idx_v)

    def body(i, carry):
        pltpu.async_copy(
            w_hbm.at[idx_v.at[pl.ds(i * _CH, _CH)]], rows, sem
        ).wait()
        pltpu.sync_copy(rows, out_hbm.at[pl.ds(base + i * _CH, _CH)])
        return carry

    lax.fori_loop(0, _NCH, body, 0)


def kernel(x, weight):
    x_flat = x.reshape(-1).astype(jnp.int32)
    out = _emb_lookup(weight, x_flat)
    return out.reshape(x.shape + (weight.shape[1],))


# speedup vs baseline: 1.8695x; 1.1095x over previous
import functools

import jax
import jax.numpy as jnp
from jax import lax
from jax.experimental import pallas as pl
from jax.experimental.pallas import tpu as pltpu
from jax.experimental.pallas import tpu_sc as plsc

_D = 64
_B = 16384 * 50
_NC = 2
_NS = 16
_NW = _NC * _NS
_BPW = _B // _NW
_CH = 128
_NCH = _BPW // _CH
_NBUF = 8
_G = _NCH // _NBUF

_mesh = plsc.VectorSubcoreMesh(core_axis_name="c", subcore_axis_name="s")


@functools.partial(
    pl.kernel,
    mesh=_mesh,
    out_type=jax.ShapeDtypeStruct((_B, _D), jnp.float32),
    scratch_types=[
        pltpu.VMEM((_BPW,), jnp.int32),
        pltpu.VMEM((_NBUF * _CH, _D), jnp.float32),
        pltpu.SemaphoreType.DMA((_NBUF,)),
        pltpu.SemaphoreType.DMA((_NBUF,)),
    ],
    compiler_params=pltpu.CompilerParams(use_tc_tiling_on_sc=False),
)
def _emb_lookup(w_hbm, x_hbm, out_hbm, idx_v, rows, gsem, ssem):
    wid = lax.axis_index("s") * _NC + lax.axis_index("c")
    base = wid * _BPW
    pltpu.sync_copy(x_hbm.at[pl.ds(base, _BPW)], idx_v)

    def gather(i, b):
        return pltpu.make_async_copy(
            w_hbm.at[idx_v.at[pl.ds(i * _CH, _CH)]],
            rows.at[pl.ds(b * _CH, _CH)],
            gsem.at[b],
        )

    def scatter(i, b):
        return pltpu.make_async_copy(
            rows.at[pl.ds(b * _CH, _CH)],
            out_hbm.at[pl.ds(base + i * _CH, _CH)],
            ssem.at[b],
        )

    for b in range(_NBUF):
        gather(b, b).start()

    def body(g, carry):
        for b in range(_NBUF):
            i = g * _NBUF + b
            gather(i, b).wait()
            scatter(i, b).start()
        for b in range(_NBUF):
            i = g * _NBUF + b
            scatter(i, b).wait()

            @pl.when(g < _G - 1)
            def _():
                gather((g + 1) * _NBUF + b, b).start()

        return carry

    lax.fori_loop(0, _G, body, 0)


def kernel(x, weight):
    x_flat = x.reshape(-1).astype(jnp.int32)
    out = _emb_lookup(weight, x_flat)
    return out.reshape(x.shape + (weight.shape[1],))


# speedup vs baseline: 2.2691x; 1.2138x over previous
import functools

import jax
import jax.numpy as jnp
from jax import lax
from jax.experimental import pallas as pl
from jax.experimental.pallas import tpu as pltpu
from jax.experimental.pallas import tpu_sc as plsc

_BR = 16384
_S = 50
_SP = 56
_D = 64
_DP = 128
_NC = 2
_NS = 16
_NW = _NC * _NS
_RPW = _BR // _NW
_NBUF = 8
_G = _RPW // _NBUF

_mesh = plsc.VectorSubcoreMesh(core_axis_name="c", subcore_axis_name="s")


@functools.partial(
    pl.kernel,
    mesh=_mesh,
    out_type=jax.ShapeDtypeStruct((_BR * _SP, _DP), jnp.float32),
    scratch_types=[
        pltpu.VMEM((_RPW, _S), jnp.int32),
        pltpu.VMEM((_NBUF * _SP, _DP), jnp.float32),
        pltpu.SemaphoreType.DMA((_NBUF,)),
        pltpu.SemaphoreType.DMA((_NBUF,)),
    ],
)
def _emb_lookup(w_hbm, x_hbm, out_hbm, idx_v, rows, gsem, ssem):
    wid = lax.axis_index("s") * _NC + lax.axis_index("c")
    rbase = wid * _RPW
    pltpu.sync_copy(x_hbm.at[pl.ds(rbase, _RPW)], idx_v)

    def gather(k, b):
        return pltpu.make_async_copy(
            w_hbm.at[idx_v.at[k]],
            rows.at[pl.ds(b * _SP, _S)],
            gsem.at[b],
        )

    def scatter(k, b):
        return pltpu.make_async_copy(
            rows.at[pl.ds(b * _SP, _SP)],
            out_hbm.at[pl.ds((rbase + k) * _SP, _SP)],
            ssem.at[b],
        )

    for b in range(_NBUF):
        gather(b, b).start()

    def body(g, carry):
        for b in range(_NBUF):
            k = g * _NBUF + b
            gather(k, b).wait()
            scatter(k, b).start()
        for b in range(_NBUF):
            k = g * _NBUF + b
            scatter(k, b).wait()

            @pl.when(g < _G - 1)
            def _():
                gather((g + 1) * _NBUF + b, b).start()

        return carry

    lax.fori_loop(0, _G, body, 0)


def kernel(x, weight):
    w_pad = jnp.pad(weight, ((0, 0), (0, _DP - _D)))
    out = _emb_lookup(w_pad, x.astype(jnp.int32))
    return out.reshape(_BR, _SP, _DP)[:, :_S, :_D]
